# baseline (speedup 1.0000x reference)
import jax
import jax.numpy as jnp
from jax import lax
from jax.experimental import pallas as pl
from jax.experimental.pallas import tpu as pltpu
from jax.experimental.pallas import tpu_sc as plsc

B, E, N, D = 256, 512, 256, 128
NC, NS = 2, 16
NW = NC * NS
BPW = B // NW
ICHUNK = 128
NCH = E // ICHUNK
LANES = 16
ZROWS = 64


def _sc_body(msg_hbm, idx_hbm, out_hbm, msg_v, idx_va, idx_vb, zero_v,
             acc_sh, sem_zero, sem_msg, sem_idx, sem_store0, sem_store1):
    cid = lax.axis_index("c")
    sid = lax.axis_index("s")
    wid = sid * NC + cid
    b0 = wid * BPW
    sem_store = (sem_store0, sem_store1)
    idx_bufs = (idx_va, idx_vb)

    def _zrow(r, _):
        for j in range(D // LANES):
            zero_v[r, pl.ds(j * LANES, LANES)] = jnp.zeros(
                (LANES,), jnp.float32)
        return _
    lax.fori_loop(0, ZROWS, _zrow, None)

    def zero_slab(p):
        slab = sid * 2 + p
        return [
            pltpu.async_copy(
                zero_v, acc_sh.at[slab, pl.ds(z * ZROWS, ZROWS)], sem_zero)
            for z in range(N // ZROWS)
        ]

    def load_chunk(i, j):
        return pltpu.async_copy(
            msg_hbm.at[b0 + i, pl.ds(j * ICHUNK, ICHUNK)], msg_v, sem_msg)

    def load_idx(i):
        return pltpu.async_copy(idx_hbm.at[b0 + i], idx_bufs[i & 1], sem_idx)

    zpend = [None, None]
    zpend[0] = zero_slab(0)
    spend = [None, None]
    ipend = load_idx(0)
    mpend = load_chunk(0, 0)

    for i in range(BPW):
        p = i & 1
        slab = sid * 2 + p
        ipend.wait()
        idx_v = idx_bufs[p]
        mpend.wait()
        for d in zpend[p]:
            d.wait()
        zpend[p] = None
        for j in range(NCH):
            pltpu.sync_copy(
                msg_v, acc_sh.at[slab].at[idx_v.at[j]], add=True)
            if j + 1 < NCH:
                pltpu.sync_copy(
                    msg_hbm.at[b0 + i, pl.ds((j + 1) * ICHUNK, ICHUNK)],
                    msg_v)
        pltpu.sync_copy(zero_v.at[0], acc_sh.at[slab, 0])
        spend[p] = pltpu.async_copy(
            acc_sh.at[slab], out_hbm.at[b0 + i], sem_store[p])
        if i + 1 < BPW:
            q = p ^ 1
            if spend[q] is not None:
                spend[q].wait()
                spend[q] = None
            zpend[q] = zero_slab(q)
            ipend = load_idx(i + 1)
            mpend = load_chunk(i + 1, 0)
    for q in (0, 1):
        if spend[q] is not None:
            spend[q].wait()


@jax.jit
def kernel(messages, tgt_idx, atom_ref):
    del atom_ref
    idx3 = tgt_idx.reshape(B, NCH, ICHUNK)
    run = pl.kernel(
        _sc_body,
        out_type=jax.ShapeDtypeStruct((B, N, D), jnp.float32),
        mesh=plsc.VectorSubcoreMesh(
            core_axis_name="c", subcore_axis_name="s"),
        scratch_types=[
            pltpu.VMEM((ICHUNK, D), jnp.float32),
            pltpu.VMEM((NCH, ICHUNK), jnp.int32),
            pltpu.VMEM((NCH, ICHUNK), jnp.int32),
            pltpu.VMEM((ZROWS, D), jnp.float32),
            pltpu.VMEM_SHARED((NS * 2, N, D), jnp.float32),
            pltpu.SemaphoreType.DMA,
            pltpu.SemaphoreType.DMA,
            pltpu.SemaphoreType.DMA,
            pltpu.SemaphoreType.DMA,
            pltpu.SemaphoreType.DMA,
        ],
    )
    return run(messages, idx3)


# speedup vs baseline: 1.1327x; 1.1327x over previous
import jax
import jax.numpy as jnp
from jax import lax
from jax.experimental import pallas as pl
from jax.experimental.pallas import tpu as pltpu
from jax.experimental.pallas import tpu_sc as plsc

B, E, N, D = 256, 512, 256, 128
NC, NS = 2, 16
NW = NC * NS
BPW = B // NW
ICHUNK = 128
NCH = E // ICHUNK
LANES = 16
ZROWS = 64


def _sc_body(msg_hbm, idx_hbm, out_hbm, msg_a, msg_b, idx_va, idx_vb,
             zero_v, acc_sh, sem_zero, sem_ma, sem_mb, sem_sa, sem_sb,
             sem_idx, sem_store0, sem_store1):
    cid = lax.axis_index("c")
    sid = lax.axis_index("s")
    wid = sid * NC + cid
    b0 = wid * BPW
    sem_store = (sem_store0, sem_store1)
    idx_bufs = (idx_va, idx_vb)

    def _zrow(r, _):
        for j in range(D // LANES):
            zero_v[r, pl.ds(j * LANES, LANES)] = jnp.zeros(
                (LANES,), jnp.float32)
        return _
    lax.fori_loop(0, ZROWS, _zrow, None)

    def zero_slab(p):
        slab = sid * 2 + p
        return [
            pltpu.async_copy(
                zero_v, acc_sh.at[slab, pl.ds(z * ZROWS, ZROWS)], sem_zero)
            for z in range(N // ZROWS)
        ]

    msg_bufs = (msg_a, msg_b)
    sem_msg = (sem_ma, sem_mb)
    sem_scat = (sem_sa, sem_sb)

    def load_pair(i, h):
        return [
            pltpu.async_copy(
                msg_hbm.at[b0 + i, pl.ds((2 * h + s) * ICHUNK, ICHUNK)],
                msg_bufs[s], sem_msg[s])
            for s in (0, 1)
        ]

    def load_idx(i):
        return pltpu.async_copy(idx_hbm.at[b0 + i], idx_bufs[i & 1], sem_idx)

    zpend = [None, None]
    zpend[0] = zero_slab(0)
    spend = [None, None]
    ipend = load_idx(0)
    mpend = load_pair(0, 0)

    for i in range(BPW):
        p = i & 1
        slab = sid * 2 + p
        ipend.wait()
        idx_v = idx_bufs[p]
        for d in zpend[p]:
            d.wait()
        zpend[p] = None
        for h in range(NCH // 2):
            for d in mpend:
                d.wait()
            scat = [
                pltpu.async_copy(
                    msg_bufs[s], acc_sh.at[slab].at[idx_v.at[2 * h + s]],
                    sem_scat[s], add=True)
                for s in (0, 1)
            ]
            for d in scat:
                d.wait()
            if h + 1 < NCH // 2:
                mpend = load_pair(i, h + 1)
        pltpu.sync_copy(zero_v.at[0], acc_sh.at[slab, 0])
        spend[p] = pltpu.async_copy(
            acc_sh.at[slab], out_hbm.at[b0 + i], sem_store[p])
        if i + 1 < BPW:
            q = p ^ 1
            if spend[q] is not None:
                spend[q].wait()
                spend[q] = None
            zpend[q] = zero_slab(q)
            ipend = load_idx(i + 1)
            mpend = load_pair(i + 1, 0)
    for q in (0, 1):
        if spend[q] is not None:
            spend[q].wait()


@jax.jit
def kernel(messages, tgt_idx, atom_ref):
    del atom_ref
    idx3 = tgt_idx.reshape(B, NCH, ICHUNK)
    run = pl.kernel(
        _sc_body,
        out_type=jax.ShapeDtypeStruct((B, N, D), jnp.float32),
        mesh=plsc.VectorSubcoreMesh(
            core_axis_name="c", subcore_axis_name="s"),
        scratch_types=[
            pltpu.VMEM((ICHUNK, D), jnp.float32),
            pltpu.VMEM((ICHUNK, D), jnp.float32),
            pltpu.VMEM((NCH, ICHUNK), jnp.int32),
            pltpu.VMEM((NCH, ICHUNK), jnp.int32),
            pltpu.VMEM((ZROWS, D), jnp.float32),
            pltpu.VMEM_SHARED((NS * 2, N, D), jnp.float32),
            pltpu.SemaphoreType.DMA,
            pltpu.SemaphoreType.DMA,
            pltpu.SemaphoreType.DMA,
            pltpu.SemaphoreType.DMA,
            pltpu.SemaphoreType.DMA,
            pltpu.SemaphoreType.DMA,
            pltpu.SemaphoreType.DMA,
            pltpu.SemaphoreType.DMA,
        ],
    )
    return run(messages, idx3)
